# Initial kernel scaffold; baseline (speedup 1.0000x reference)
import jax
import jax.numpy as jnp
from jax.experimental import pallas as pl


def kernel(positions, pe):
    raise NotImplementedError("write your pallas kernel here")


# speedup vs baseline: 1.6170x; 1.6170x over previous
import jax
import jax.numpy as jnp
from jax import lax
from jax.experimental import pallas as pl
from jax.experimental.pallas import tpu as pltpu
from jax.experimental.pallas import tpu_sc as plsc

_D = 2048
_NC = 2
_NS = 16
_NW = _NC * _NS
_CHUNK = 16


def _pe_gather(pos_hbm, pe_hbm, out_hbm, idx_v, buf, sem):
    b_per_w = pos_hbm.shape[0] // _NW
    n_chunks = b_per_w // _CHUNK
    wid = lax.axis_index("s") * _NC + lax.axis_index("c")
    base = wid * b_per_w
    pltpu.sync_copy(pos_hbm.at[pl.ds(base, b_per_w)], idx_v)

    def body(c, carry):
        off = c * _CHUNK
        pltpu.async_copy(pe_hbm.at[idx_v.at[pl.ds(off, _CHUNK)]], buf, sem).wait()
        pltpu.sync_copy(buf, out_hbm.at[pl.ds(base + off, _CHUNK)])
        return carry

    lax.fori_loop(0, n_chunks, body, 0)


def kernel(positions, pe):
    b, s = positions.shape
    d = pe.shape[1]
    flat = positions.reshape(b * s)
    mesh = plsc.VectorSubcoreMesh(core_axis_name="c", subcore_axis_name="s")
    out = pl.kernel(
        _pe_gather,
        out_type=jax.ShapeDtypeStruct((b * s, d), jnp.float32),
        mesh=mesh,
        scratch_types=[
            pltpu.VMEM((b * s // _NW,), jnp.int32),
            pltpu.VMEM((_CHUNK, d), jnp.float32),
            pltpu.SemaphoreType.DMA,
        ],
    )(flat, pe)
    return out.reshape(b, s, d)


# speedup vs baseline: 1.9411x; 1.2004x over previous
import jax
import jax.numpy as jnp
from jax import lax
from jax.experimental import pallas as pl
from jax.experimental.pallas import tpu as pltpu
from jax.experimental.pallas import tpu_sc as plsc

_D = 2048
_NC = 2
_NS = 16
_NW = _NC * _NS
_CHUNK = 16


def _pe_gather(pos_hbm, pe_hbm, out_hbm, idx_v, buf0, buf1, sem0, sem1):
    b_per_w = pos_hbm.shape[0] // _NW
    n_pairs = b_per_w // (2 * _CHUNK)
    wid = lax.axis_index("s") * _NC + lax.axis_index("c")
    base = wid * b_per_w
    pltpu.sync_copy(pos_hbm.at[pl.ds(base, b_per_w)], idx_v)

    def gather(chunk, buf, sem):
        off = chunk * _CHUNK
        return pltpu.async_copy(
            pe_hbm.at[idx_v.at[pl.ds(off, _CHUNK)]], buf, sem)

    gather(0, buf0, sem0)
    gather(1, buf1, sem1)

    def body(j, carry):
        c0 = 2 * j
        pltpu.make_async_copy(pe_hbm.at[pl.ds(0, _CHUNK)], buf0, sem0).wait()
        pltpu.sync_copy(buf0, out_hbm.at[pl.ds(base + c0 * _CHUNK, _CHUNK)])

        @pl.when(j < n_pairs - 1)
        def _():
            gather(c0 + 2, buf0, sem0)

        pltpu.make_async_copy(pe_hbm.at[pl.ds(0, _CHUNK)], buf1, sem1).wait()
        pltpu.sync_copy(buf1, out_hbm.at[pl.ds(base + (c0 + 1) * _CHUNK, _CHUNK)])

        @pl.when(j < n_pairs - 1)
        def _():
            gather(c0 + 3, buf1, sem1)

        return carry

    lax.fori_loop(0, n_pairs, body, 0)


def kernel(positions, pe):
    b, s = positions.shape
    d = pe.shape[1]
    flat = positions.reshape(b * s)
    mesh = plsc.VectorSubcoreMesh(core_axis_name="c", subcore_axis_name="s")
    out = pl.kernel(
        _pe_gather,
        out_type=jax.ShapeDtypeStruct((b * s, d), jnp.float32),
        mesh=mesh,
        scratch_types=[
            pltpu.VMEM((b * s // _NW,), jnp.int32),
            pltpu.VMEM((_CHUNK, d), jnp.float32),
            pltpu.VMEM((_CHUNK, d), jnp.float32),
            pltpu.SemaphoreType.DMA,
            pltpu.SemaphoreType.DMA,
        ],
    )(flat, pe)
    return out.reshape(b, s, d)


# speedup vs baseline: 1.9449x; 1.0020x over previous
import jax
import jax.numpy as jnp
from jax import lax
from jax.experimental import pallas as pl
from jax.experimental.pallas import tpu as pltpu
from jax.experimental.pallas import tpu_sc as plsc

_NC = 2
_NS = 16
_NW = _NC * _NS
_CHUNK = 8
_NBUF = 4


def _pe_gather(pos_hbm, pe_hbm, out_hbm, idx_v, bufs, gsems, ssems):
    b_per_w = pos_hbm.shape[0] // _NW
    n_chunks = b_per_w // _CHUNK
    wid = lax.axis_index("s") * _NC + lax.axis_index("c")
    base = wid * b_per_w
    pltpu.sync_copy(pos_hbm.at[pl.ds(base, b_per_w)], idx_v)

    def gather_start(chunk, b):
        off = chunk * _CHUNK
        pltpu.async_copy(
            pe_hbm.at[idx_v.at[pl.ds(off, _CHUNK)]], bufs[b], gsems[b])

    def gather_wait(b):
        pltpu.make_async_copy(
            pe_hbm.at[pl.ds(0, _CHUNK)], bufs[b], gsems[b]).wait()

    def store_start(chunk, b):
        pltpu.async_copy(
            bufs[b], out_hbm.at[pl.ds(base + chunk * _CHUNK, _CHUNK)],
            ssems[b])

    def store_wait(b):
        pltpu.make_async_copy(
            bufs[b], out_hbm.at[pl.ds(base, _CHUNK)], ssems[b]).wait()

    gather_start(0, 0)
    gather_start(1, 1)

    def body(j, carry):
        for b in range(_NBUF):
            c = _NBUF * j + b

            @pl.when(c >= 2)
            def _():
                store_wait((b + 2) % _NBUF)

            @pl.when(c + 2 < n_chunks)
            def _():
                gather_start(c + 2, (b + 2) % _NBUF)

            gather_wait(b)
            store_start(c, b)
        return carry

    lax.fori_loop(0, n_chunks // _NBUF, body, 0)
    store_wait((n_chunks - 2) % _NBUF)
    store_wait((n_chunks - 1) % _NBUF)


def kernel(positions, pe):
    b, s = positions.shape
    d = pe.shape[1]
    flat = positions.reshape(b * s)
    mesh = plsc.VectorSubcoreMesh(core_axis_name="c", subcore_axis_name="s")

    def body(pos_hbm, pe_hbm, out_hbm, idx_v, b0, b1, b2, b3,
             g0, g1, g2, g3, s0, s1, s2, s3):
        _pe_gather(pos_hbm, pe_hbm, out_hbm, idx_v,
                   (b0, b1, b2, b3), (g0, g1, g2, g3), (s0, s1, s2, s3))

    out = pl.kernel(
        body,
        out_type=jax.ShapeDtypeStruct((b * s, d), jnp.float32),
        mesh=mesh,
        scratch_types=(
            [pltpu.VMEM((b * s // _NW,), jnp.int32)]
            + [pltpu.VMEM((_CHUNK, d), jnp.float32)] * _NBUF
            + [pltpu.SemaphoreType.DMA] * (2 * _NBUF)
        ),
    )(flat, pe)
    return out.reshape(b, s, d)
